# baseline (speedup 1.0000x reference)
import functools

import jax
import jax.numpy as jnp
from jax import lax
from jax.experimental import pallas as pl
from jax.experimental.pallas import tpu as pltpu
from jax.experimental.pallas import tpu_sc as plsc

_N = 1000
_D = 4096
_CHUNK_ROWS = 8
_NW = 32

_K = 352
_CSTART = _K // _CHUNK_ROWS
_NCHUNKS = _N // _CHUNK_ROWS

_mesh = plsc.VectorSubcoreMesh(core_axis_name="c", subcore_axis_name="s")


@functools.partial(
    pl.kernel,
    mesh=_mesh,
    out_type=jax.ShapeDtypeStruct((_N, _D), jnp.float32),
    scratch_types=[
        pltpu.VMEM((_CHUNK_ROWS, _D), jnp.float32),
        pltpu.VMEM((_CHUNK_ROWS, _D), jnp.float32),
        pltpu.VMEM((_CHUNK_ROWS, _D), jnp.float32),
        pltpu.SemaphoreType.DMA,
        pltpu.SemaphoreType.DMA,
        pltpu.SemaphoreType.DMA,
        pltpu.SemaphoreType.DMA,
        pltpu.SemaphoreType.DMA,
        pltpu.SemaphoreType.DMA,
    ],
)
def _sc_copy(src_hbm, out_hbm, b0, b1, b2, si0, si1, si2, so0, so1, so2):
    wid = lax.axis_index("s") * 2 + lax.axis_index("c")
    bufs = (b0, b1, b2)
    sins = (si0, si1, si2)
    souts = (so0, so1, so2)

    def refs(i):
        r = (_CSTART + wid + _NW * i) * _CHUNK_ROWS
        return src_hbm.at[pl.ds(r, _CHUNK_ROWS), :], out_hbm.at[pl.ds(r, _CHUNK_ROWS), :]

    cond2 = _CSTART + wid + _NW * 2 < _NCHUNKS
    s0, d0 = refs(0)
    s1, d1 = refs(1)
    s2, d2 = refs(2)
    h0 = pltpu.async_copy(s0, b0, si0)
    h1 = pltpu.async_copy(s1, b1, si1)

    @pl.when(cond2)
    def _():
        pltpu.async_copy(s2, b2, si2)

    h0.wait()
    o0 = pltpu.async_copy(b0, d0, so0)
    h1.wait()
    o1 = pltpu.async_copy(b1, d1, so1)

    @pl.when(cond2)
    def _():
        pltpu.make_async_copy(s2, b2, si2).wait()
        pltpu.async_copy(b2, d2, so2)

    o0.wait()
    o1.wait()

    @pl.when(cond2)
    def _():
        pltpu.make_async_copy(b2, d2, so2).wait()


_TCB = 32


def _tc_body(in_ref, out_ref):
    out_ref[...] = in_ref[...]


_tc_copy = pl.pallas_call(
    _tc_body,
    grid=(_K // _TCB,),
    in_specs=[pl.BlockSpec((_TCB, _D), lambda i: (i, 0))],
    out_specs=pl.BlockSpec((_TCB, _D), lambda i: (i, 0)),
    out_shape=jax.ShapeDtypeStruct((_K, _D), jnp.float32),
)


def kernel(special_embeddings_weight):
    sc_full = _sc_copy(special_embeddings_weight)
    tc_part = _tc_copy(special_embeddings_weight)
    return lax.dynamic_update_slice(sc_full, tc_part, (0, 0))


# speedup vs baseline: 1.1294x; 1.1294x over previous
import functools

import jax
import jax.numpy as jnp
from jax import lax
from jax.experimental import pallas as pl
from jax.experimental.pallas import tpu as pltpu
from jax.experimental.pallas import tpu_sc as plsc

_N = 1000
_D = 4096
_CHUNK_ROWS = 8
_NCHUNKS = _N // _CHUNK_ROWS
_NW = 32
_ITERS = -(-_NCHUNKS // _NW)

_mesh = plsc.VectorSubcoreMesh(core_axis_name="c", subcore_axis_name="s")


@functools.partial(
    pl.kernel,
    mesh=_mesh,
    out_type=jax.ShapeDtypeStruct((_N, _D), jnp.float32),
    scratch_types=[
        pltpu.VMEM((_CHUNK_ROWS, _D), jnp.float32),
        pltpu.VMEM((_CHUNK_ROWS, _D), jnp.float32),
        pltpu.VMEM((_CHUNK_ROWS, _D), jnp.float32),
        pltpu.SemaphoreType.DMA,
        pltpu.SemaphoreType.DMA,
        pltpu.SemaphoreType.DMA,
        pltpu.SemaphoreType.DMA,
        pltpu.SemaphoreType.DMA,
        pltpu.SemaphoreType.DMA,
    ],
)
def _copy_kernel(src_hbm, out_hbm, b0, b1, b2, si0, si1, si2, so0, so1, so2):
    wid = lax.axis_index("s") * 2 + lax.axis_index("c")
    bufs = (b0, b1, b2)
    sins = (si0, si1, si2)
    souts = (so0, so1, so2)

    def start_in(i):
        r = (wid + _NW * i) * _CHUNK_ROWS
        return pltpu.async_copy(
            src_hbm.at[pl.ds(r, _CHUNK_ROWS), :], bufs[i % 3], sins[i % 3])

    def start_out(i):
        r = (wid + _NW * i) * _CHUNK_ROWS
        return pltpu.async_copy(
            bufs[i % 3], out_hbm.at[pl.ds(r, _CHUNK_ROWS), :], souts[i % 3])

    h_in0 = start_in(0)
    h_in1 = start_in(1)
    h_in2 = start_in(2)
    h_in0.wait()
    h_out0 = start_out(0)
    h_in1.wait()
    h_out1 = start_out(1)
    h_in2.wait()
    h_out2 = start_out(2)
    h_out0.wait()

    @pl.when(wid + _NW * 3 < _NCHUNKS)
    def _():
        h_in3 = start_in(3)
        h_in3.wait()
        h_out3 = start_out(3)
        h_out3.wait()

    h_out1.wait()
    h_out2.wait()


def kernel(special_embeddings_weight):
    return _copy_kernel(special_embeddings_weight)
